# Initial kernel scaffold; baseline (speedup 1.0000x reference)
import jax
import jax.numpy as jnp
from jax.experimental import pallas as pl


def kernel(movie_id, padded_genre_indices, year_idx, padded_tag_indices, movie_table, genre_table, tag_table, year_table, W1, b1, W2, b2, W3, b3):
    raise NotImplementedError("write your pallas kernel here")


# speedup vs baseline: 2.6663x; 2.6663x over previous
import functools

import jax
import jax.numpy as jnp
from jax import lax
from jax.experimental import pallas as pl
from jax.experimental.pallas import tpu as pltpu
from jax.experimental.pallas import tpu_sc as plsc

NC = 2
NS = 16
NW = NC * NS
LANES = 16

D = 64
YD = 16
NG = 8
NT = 20

CHUNK = 64


def _accumulate(rows_ref, acc_ref, n_per, n_samples):
  def body(i, carry):
    r0 = i * n_per
    for cc in range(D // LANES):
      v = rows_ref[r0, pl.ds(cc * LANES, LANES)]
      for j in range(1, n_per):
        v = v + rows_ref[r0 + j, pl.ds(cc * LANES, LANES)]
      acc_ref[i, pl.ds(cc * LANES, LANES)] = v
    return carry

  lax.fori_loop(0, n_samples, body, 0)


def _sc_gather_pool(B):
  bpw = B // NW
  nch = bpw // CHUNK

  mesh = plsc.VectorSubcoreMesh(core_axis_name="c", subcore_axis_name="s")

  @functools.partial(
      pl.kernel,
      mesh=mesh,
      compiler_params=pltpu.CompilerParams(use_tc_tiling_on_sc=False),
      out_type=[
          jax.ShapeDtypeStruct((B, D), jnp.float32),
          jax.ShapeDtypeStruct((B, D), jnp.float32),
          jax.ShapeDtypeStruct((B, YD), jnp.float32),
          jax.ShapeDtypeStruct((B, D), jnp.float32),
      ],
      scratch_types=[
          pltpu.VMEM((CHUNK * NT, D), jnp.float32),
          pltpu.VMEM((CHUNK, D), jnp.float32),
          pltpu.VMEM((CHUNK, YD), jnp.float32),
          pltpu.VMEM((CHUNK, D), jnp.float32),
          pltpu.VMEM((CHUNK, D), jnp.float32),
          pltpu.VMEM((CHUNK * NT,), jnp.int32),
          pltpu.VMEM((CHUNK * NG,), jnp.int32),
          pltpu.VMEM((CHUNK,), jnp.int32),
          pltpu.VMEM((CHUNK,), jnp.int32),
          pltpu.SemaphoreType.DMA,
      ],
  )
  def sc_kernel(movie_id, g_idx_flat, year_idx, t_idx_flat,
                movie_table, genre_table, tag_table, year_table,
                m_out, g_out, y_out, t_out,
                rows, m_rows, y_rows, g_acc, t_acc,
                idx_t, idx_g, idx_m, idx_y, sem):
    wid = lax.axis_index("s") * NC + lax.axis_index("c")
    base = wid * bpw

    def chunk_body(c, carry):
      start = base + c * CHUNK

      pltpu.sync_copy(movie_id.at[pl.ds(start, CHUNK)], idx_m)
      pltpu.async_copy(movie_table.at[idx_m], m_rows, sem).wait()
      pltpu.sync_copy(m_rows, m_out.at[pl.ds(start, CHUNK)])

      pltpu.sync_copy(year_idx.at[pl.ds(start, CHUNK)], idx_y)
      pltpu.async_copy(year_table.at[idx_y], y_rows, sem).wait()
      pltpu.sync_copy(y_rows, y_out.at[pl.ds(start, CHUNK)])

      pltpu.sync_copy(g_idx_flat.at[pl.ds(start * NG, CHUNK * NG)], idx_g)
      hs = []
      for g in range(CHUNK * NG // 128):
        hs.append(pltpu.async_copy(
            genre_table.at[idx_g.at[pl.ds(g * 128, 128)]],
            rows.at[pl.ds(g * 128, 128), :], sem))
      for h in hs:
        h.wait()
      _accumulate(rows, g_acc, NG, CHUNK)
      pltpu.sync_copy(g_acc, g_out.at[pl.ds(start, CHUNK)])

      pltpu.sync_copy(t_idx_flat.at[pl.ds(start * NT, CHUNK * NT)], idx_t)
      hs = []
      for g in range(CHUNK * NT // 128):
        hs.append(pltpu.async_copy(
            tag_table.at[idx_t.at[pl.ds(g * 128, 128)]],
            rows.at[pl.ds(g * 128, 128), :], sem))
      for h in hs:
        h.wait()
      _accumulate(rows, t_acc, NT, CHUNK)
      pltpu.sync_copy(t_acc, t_out.at[pl.ds(start, CHUNK)])
      return carry

    lax.fori_loop(0, nch, chunk_body, 0)

  return sc_kernel


def _mlp_body(gidx_ref, tidx_ref, m_ref, g_ref, y_ref, t_ref,
              w1a_ref, w1b_ref, w1c_ref, w1d_ref, b1_ref,
              w2_ref, b2_ref, w3_ref, b3_ref, out_ref):
  cnt_g = jnp.sum((gidx_ref[...] != 0).astype(jnp.float32), axis=1,
                  keepdims=True)
  cnt_t = jnp.sum((tidx_ref[...] != 0).astype(jnp.float32), axis=1,
                  keepdims=True)
  g = g_ref[...] / jnp.maximum(cnt_g, 1e-9)
  t = t_ref[...] / jnp.maximum(cnt_t, 1e-9)
  f32 = jnp.float32
  x = (jnp.dot(m_ref[...], w1a_ref[...], preferred_element_type=f32)
       + jnp.dot(g, w1b_ref[...], preferred_element_type=f32)
       + jnp.dot(y_ref[...], w1c_ref[...], preferred_element_type=f32)
       + jnp.dot(t, w1d_ref[...], preferred_element_type=f32)
       + b1_ref[...])
  x = jnp.maximum(x, 0.0)
  x = jnp.maximum(jnp.dot(x, w2_ref[...], preferred_element_type=f32)
                  + b2_ref[...], 0.0)
  out_ref[...] = jnp.dot(x, w3_ref[...], preferred_element_type=f32) + b3_ref[...]


def _mlp(B, H, gidx, tidx, m, g_sum, y, t_sum, W1, b1, W2, b2, W3, b3):
  R = 2048
  grid = B // R
  w1a = W1[:D]
  w1b = W1[D:2 * D]
  w1c = W1[2 * D:2 * D + YD]
  w1d = W1[2 * D + YD:]
  row = lambda i: (i, 0)
  rep = lambda i: (0, 0)
  return pl.pallas_call(
      _mlp_body,
      grid=(grid,),
      in_specs=[
          pl.BlockSpec((R, NG), row),
          pl.BlockSpec((R, NT), row),
          pl.BlockSpec((R, D), row),
          pl.BlockSpec((R, D), row),
          pl.BlockSpec((R, YD), row),
          pl.BlockSpec((R, D), row),
          pl.BlockSpec(w1a.shape, rep),
          pl.BlockSpec(w1b.shape, rep),
          pl.BlockSpec(w1c.shape, rep),
          pl.BlockSpec(w1d.shape, rep),
          pl.BlockSpec((1, H), rep),
          pl.BlockSpec(W2.shape, rep),
          pl.BlockSpec((1, H // 2), rep),
          pl.BlockSpec(W3.shape, rep),
          pl.BlockSpec((1, D), rep),
      ],
      out_specs=pl.BlockSpec((R, D), row),
      out_shape=jax.ShapeDtypeStruct((B, D), jnp.float32),
  )(gidx, tidx, m, g_sum, y, t_sum, w1a, w1b, w1c, w1d,
    b1.reshape(1, H), W2, b2.reshape(1, H // 2), W3, b3.reshape(1, D))


def kernel(movie_id, padded_genre_indices, year_idx, padded_tag_indices,
           movie_table, genre_table, tag_table, year_table,
           W1, b1, W2, b2, W3, b3):
  B = movie_id.shape[0]
  H = W1.shape[1]
  mid = movie_id.astype(jnp.int32)
  gid = padded_genre_indices.astype(jnp.int32).reshape(-1)
  yid = year_idx.astype(jnp.int32)
  tid = padded_tag_indices.astype(jnp.int32).reshape(-1)

  m_rows, g_sum, y_rows, t_sum = _sc_gather_pool(B)(
      mid, gid, yid, tid, movie_table, genre_table, tag_table, year_table)

  return _mlp(B, H, padded_genre_indices.astype(jnp.int32),
              padded_tag_indices.astype(jnp.int32),
              m_rows, g_sum, y_rows, t_sum, W1, b1, W2, b2, W3, b3)


# speedup vs baseline: 5.4331x; 2.0377x over previous
import functools

import jax
import jax.numpy as jnp
from jax import lax
from jax.experimental import pallas as pl
from jax.experimental.pallas import tpu as pltpu
from jax.experimental.pallas import tpu_sc as plsc

NC = 2
NS = 16
NW = NC * NS
LANES = 16

D = 64
YD = 16
NG = 8
NT = 20

CHUNK = 64
CONV_CB = 8192

M_GRID = 62
T_GRID = 7
M_SPLIT = CONV_CB * M_GRID
T_SPLIT = CONV_CB * T_GRID


def _conv_body(a_ref, b_ref, eye_ref, out_ref):
  f32 = jnp.float32
  dn = (((0,), (0,)), ((), ()))
  at = lax.dot_general(a_ref[...], eye_ref[...], dimension_numbers=dn,
                       preferred_element_type=f32)
  bt = lax.dot_general(b_ref[...], eye_ref[...], dimension_numbers=dn,
                       preferred_element_type=f32)
  out_ref[...] = jnp.concatenate([at, bt], axis=1)


def _table_convert(table_t, grid):
  split = CONV_CB * grid
  n = table_t.shape[1]
  last_blk = (n - 1) // CONV_CB
  eye = jnp.eye(D, dtype=jnp.float32)
  out = pl.pallas_call(
      _conv_body,
      grid=(grid,),
      in_specs=[
          pl.BlockSpec((D, CONV_CB), lambda i: (0, i)),
          pl.BlockSpec((D, CONV_CB),
                       lambda i, g=grid, lb=last_blk:
                       (0, jnp.minimum(i + g, lb))),
          pl.BlockSpec((D, D), lambda i: (0, 0)),
      ],
      out_specs=pl.BlockSpec((CONV_CB, 2 * D), lambda i: (i, 0)),
      out_shape=jax.ShapeDtypeStruct((split, 2 * D), jnp.float32),
  )(table_t, table_t, eye)
  return out.reshape(2 * split, D)


def _remap(v, split):
  v2 = v + v
  return jnp.where(v < split, v2, v2 - 2 * split + 1)


def _accumulate(rows_ref, acc_ref, n_per, n_samples):
  def body(i, carry):
    for cc in range(D // LANES):
      v = rows_ref[i, pl.ds(cc * LANES, LANES)]
      for j in range(1, n_per):
        v = v + rows_ref[j * n_samples + i, pl.ds(cc * LANES, LANES)]
      acc_ref[i, pl.ds(cc * LANES, LANES)] = v
    return carry

  lax.fori_loop(0, n_samples, body, 0)


def _sc_gather_pool(B):
  bpw = B // NW
  nch = bpw // CHUNK

  mesh = plsc.VectorSubcoreMesh(core_axis_name="c", subcore_axis_name="s")

  @functools.partial(
      pl.kernel,
      mesh=mesh,
      compiler_params=pltpu.CompilerParams(use_tc_tiling_on_sc=False),
      out_type=[
          jax.ShapeDtypeStruct((B, D), jnp.float32),
          jax.ShapeDtypeStruct((B, D), jnp.float32),
          jax.ShapeDtypeStruct((B, YD), jnp.float32),
          jax.ShapeDtypeStruct((B, D), jnp.float32),
      ],
      scratch_types=[
          pltpu.VMEM((CHUNK * NT, D), jnp.float32),
          pltpu.VMEM((CHUNK, D), jnp.float32),
          pltpu.VMEM((CHUNK, YD), jnp.float32),
          pltpu.VMEM((CHUNK, D), jnp.float32),
          pltpu.VMEM((CHUNK, D), jnp.float32),
          pltpu.VMEM((NT, bpw), jnp.int32),
          pltpu.VMEM((NG, bpw), jnp.int32),
          pltpu.VMEM((bpw,), jnp.int32),
          pltpu.VMEM((bpw,), jnp.int32),
          pltpu.SemaphoreType.DMA,
          pltpu.SemaphoreType.DMA,
      ],
  )
  def sc_kernel(movie_id, g_idx_t, year_idx, t_idx_t,
                movie_lin, genre_table, tag_lin, year_table,
                m_out, g_out, y_out, t_out,
                rows, m_rows, y_rows, g_acc, t_acc,
                idx_t, idx_g, idx_m, idx_y, sem, sem2):
    wid = lax.axis_index("s") * NC + lax.axis_index("c")
    base = wid * bpw

    pltpu.sync_copy(movie_id.at[pl.ds(base, bpw)], idx_m)
    pltpu.sync_copy(year_idx.at[pl.ds(base, bpw)], idx_y)
    pltpu.sync_copy(g_idx_t.at[:, pl.ds(base, bpw)], idx_g)
    pltpu.sync_copy(t_idx_t.at[:, pl.ds(base, bpw)], idx_t)

    def chunk_body(c, carry):
      off = c * CHUNK
      start = base + off

      hm = pltpu.async_copy(
          movie_lin.at[idx_m.at[pl.ds(off, CHUNK)]], m_rows, sem2)
      pltpu.async_copy(year_table.at[idx_y.at[pl.ds(off, CHUNK)]],
                       y_rows, sem).wait()
      pltpu.sync_copy(y_rows, y_out.at[pl.ds(start, CHUNK)])
      hm.wait()
      pltpu.sync_copy(m_rows, m_out.at[pl.ds(start, CHUNK)])

      hs = []
      for j in range(NG):
        hs.append(pltpu.async_copy(
            genre_table.at[idx_g.at[j, pl.ds(off, CHUNK)]],
            rows.at[pl.ds(j * CHUNK, CHUNK), :], sem))
      for h in hs:
        h.wait()
      _accumulate(rows, g_acc, NG, CHUNK)
      pltpu.sync_copy(g_acc, g_out.at[pl.ds(start, CHUNK)])

      hs = []
      for j in range(NT):
        hs.append(pltpu.async_copy(
            tag_lin.at[idx_t.at[j, pl.ds(off, CHUNK)]],
            rows.at[pl.ds(j * CHUNK, CHUNK), :], sem))
      for h in hs:
        h.wait()
      _accumulate(rows, t_acc, NT, CHUNK)
      pltpu.sync_copy(t_acc, t_out.at[pl.ds(start, CHUNK)])
      return carry

    lax.fori_loop(0, nch, chunk_body, 0)

  return sc_kernel


def _mlp_body(gidx_ref, tidx_ref, m_ref, g_ref, y_ref, t_ref,
              w1a_ref, w1b_ref, w1c_ref, w1d_ref, b1_ref,
              w2_ref, b2_ref, w3_ref, b3_ref, out_ref):
  cnt_g = jnp.sum((gidx_ref[...] != 0).astype(jnp.float32), axis=1,
                  keepdims=True)
  cnt_t = jnp.sum((tidx_ref[...] != 0).astype(jnp.float32), axis=1,
                  keepdims=True)
  g = g_ref[...] / jnp.maximum(cnt_g, 1e-9)
  t = t_ref[...] / jnp.maximum(cnt_t, 1e-9)
  f32 = jnp.float32
  x = (jnp.dot(m_ref[...], w1a_ref[...], preferred_element_type=f32)
       + jnp.dot(g, w1b_ref[...], preferred_element_type=f32)
       + jnp.dot(y_ref[...], w1c_ref[...], preferred_element_type=f32)
       + jnp.dot(t, w1d_ref[...], preferred_element_type=f32)
       + b1_ref[...])
  x = jnp.maximum(x, 0.0)
  x = jnp.maximum(jnp.dot(x, w2_ref[...], preferred_element_type=f32)
                  + b2_ref[...], 0.0)
  out_ref[...] = jnp.dot(x, w3_ref[...], preferred_element_type=f32) + b3_ref[...]


def _mlp(B, H, gidx, tidx, m, g_sum, y, t_sum, W1, b1, W2, b2, W3, b3):
  R = 2048
  grid = B // R
  w1a = W1[:D]
  w1b = W1[D:2 * D]
  w1c = W1[2 * D:2 * D + YD]
  w1d = W1[2 * D + YD:]
  row = lambda i: (i, 0)
  rep = lambda i: (0, 0)
  return pl.pallas_call(
      _mlp_body,
      grid=(grid,),
      in_specs=[
          pl.BlockSpec((R, NG), row),
          pl.BlockSpec((R, NT), row),
          pl.BlockSpec((R, D), row),
          pl.BlockSpec((R, D), row),
          pl.BlockSpec((R, YD), row),
          pl.BlockSpec((R, D), row),
          pl.BlockSpec(w1a.shape, rep),
          pl.BlockSpec(w1b.shape, rep),
          pl.BlockSpec(w1c.shape, rep),
          pl.BlockSpec(w1d.shape, rep),
          pl.BlockSpec((1, H), rep),
          pl.BlockSpec(W2.shape, rep),
          pl.BlockSpec((1, H // 2), rep),
          pl.BlockSpec(W3.shape, rep),
          pl.BlockSpec((1, D), rep),
      ],
      out_specs=pl.BlockSpec((R, D), row),
      out_shape=jax.ShapeDtypeStruct((B, D), jnp.float32),
  )(gidx, tidx, m, g_sum, y, t_sum, w1a, w1b, w1c, w1d,
    b1.reshape(1, H), W2, b2.reshape(1, H // 2), W3, b3.reshape(1, D))


def kernel(movie_id, padded_genre_indices, year_idx, padded_tag_indices,
           movie_table, genre_table, tag_table, year_table,
           W1, b1, W2, b2, W3, b3):
  B = movie_id.shape[0]
  H = W1.shape[1]
  mid = movie_id.astype(jnp.int32)
  gid_t = padded_genre_indices.astype(jnp.int32).T
  yid = year_idx.astype(jnp.int32)
  mid_lin = _remap(mid, M_SPLIT)
  tid_lin_t = _remap(padded_tag_indices.astype(jnp.int32), T_SPLIT).T

  movie_lin = _table_convert(movie_table.T, M_GRID)
  tag_lin = _table_convert(tag_table.T, T_GRID)

  m_rows, g_sum, y_rows, t_sum = _sc_gather_pool(B)(
      mid_lin, gid_t, yid, tid_lin_t, movie_lin, genre_table, tag_lin,
      year_table)

  return _mlp(B, H, padded_genre_indices.astype(jnp.int32),
              padded_tag_indices.astype(jnp.int32),
              m_rows, g_sum, y_rows, t_sum, W1, b1, W2, b2, W3, b3)


# speedup vs baseline: 5.6646x; 1.0426x over previous
import functools

import jax
import jax.numpy as jnp
from jax import lax
from jax.experimental import pallas as pl
from jax.experimental.pallas import tpu as pltpu
from jax.experimental.pallas import tpu_sc as plsc

NC = 2
NS = 16
NW = NC * NS
LANES = 16

D = 64
YD = 16
NG = 8
NT = 20

CHUNK = 64
CONV_CB = 8192

M_GRID = 62
T_GRID = 7
M_SPLIT = CONV_CB * M_GRID
T_SPLIT = CONV_CB * T_GRID


def _conv_body(a_ref, b_ref, eye_ref, out_ref):
  f32 = jnp.float32
  dn = (((0,), (0,)), ((), ()))
  at = lax.dot_general(a_ref[...], eye_ref[...], dimension_numbers=dn,
                       preferred_element_type=f32)
  bt = lax.dot_general(b_ref[...], eye_ref[...], dimension_numbers=dn,
                       preferred_element_type=f32)
  out_ref[...] = jnp.concatenate([at, bt], axis=1)


def _table_convert(table_t, grid):
  split = CONV_CB * grid
  n = table_t.shape[1]
  last_blk = (n - 1) // CONV_CB
  eye = jnp.eye(D, dtype=jnp.float32)
  out = pl.pallas_call(
      _conv_body,
      grid=(grid,),
      in_specs=[
          pl.BlockSpec((D, CONV_CB), lambda i: (0, i)),
          pl.BlockSpec((D, CONV_CB),
                       lambda i, g=grid, lb=last_blk:
                       (0, jnp.minimum(i + g, lb))),
          pl.BlockSpec((D, D), lambda i: (0, 0)),
      ],
      out_specs=pl.BlockSpec((CONV_CB, 2 * D), lambda i: (i, 0)),
      out_shape=jax.ShapeDtypeStruct((split, 2 * D), jnp.float32),
  )(table_t, table_t, eye)
  return out.reshape(2 * split, D)


def _remap(v, split):
  v2 = v + v
  return jnp.where(v < split, v2, v2 - 2 * split + 1)


def _accumulate(rows_ref, acc_ref, n_per, n_samples):
  def body(i, carry):
    for cc in range(D // LANES):
      v = rows_ref[i, pl.ds(cc * LANES, LANES)]
      for j in range(1, n_per):
        v = v + rows_ref[j * n_samples + i, pl.ds(cc * LANES, LANES)]
      acc_ref[i, pl.ds(cc * LANES, LANES)] = v
    return carry

  lax.fori_loop(0, n_samples, body, 0)


def _sc_rows(B):
  bpw = B // NW
  mesh = plsc.VectorSubcoreMesh(core_axis_name="c", subcore_axis_name="s")

  @functools.partial(
      pl.kernel,
      mesh=mesh,
      compiler_params=pltpu.CompilerParams(use_tc_tiling_on_sc=False),
      out_type=[
          jax.ShapeDtypeStruct((B, D), jnp.float32),
          jax.ShapeDtypeStruct((B, YD), jnp.float32),
      ],
      scratch_types=[
          pltpu.VMEM((B // NW, D), jnp.float32),
          pltpu.VMEM((B // NW, YD), jnp.float32),
          pltpu.VMEM((B // NW,), jnp.int32),
          pltpu.VMEM((B // NW,), jnp.int32),
          pltpu.SemaphoreType.DMA,
          pltpu.SemaphoreType.DMA,
      ],
  )
  def sc_kernel(movie_id, year_idx, movie_lin, year_table,
                m_out, y_out, m_rows, y_rows, idx_m, idx_y, sem, sem2):
    wid = lax.axis_index("s") * NC + lax.axis_index("c")
    base = wid * bpw
    pltpu.sync_copy(movie_id.at[pl.ds(base, bpw)], idx_m)
    pltpu.sync_copy(year_idx.at[pl.ds(base, bpw)], idx_y)
    hs = []
    for g in range(bpw // 128):
      hs.append(pltpu.async_copy(
          movie_lin.at[idx_m.at[pl.ds(g * 128, 128)]],
          m_rows.at[pl.ds(g * 128, 128), :], sem))
      hs.append(pltpu.async_copy(
          year_table.at[idx_y.at[pl.ds(g * 128, 128)]],
          y_rows.at[pl.ds(g * 128, 128), :], sem2))
    for h in hs:
      h.wait()
    pltpu.sync_copy(m_rows, m_out.at[pl.ds(base, bpw)])
    pltpu.sync_copy(y_rows, y_out.at[pl.ds(base, bpw)])

  return sc_kernel


def _sc_gather_pool(B):
  bpw = B // NW
  nch = bpw // CHUNK

  mesh = plsc.VectorSubcoreMesh(core_axis_name="c", subcore_axis_name="s")

  @functools.partial(
      pl.kernel,
      mesh=mesh,
      compiler_params=pltpu.CompilerParams(use_tc_tiling_on_sc=False),
      out_type=[
          jax.ShapeDtypeStruct((B, D), jnp.float32),
          jax.ShapeDtypeStruct((B, D), jnp.float32),
      ],
      scratch_types=[
          pltpu.VMEM((CHUNK * NT, D), jnp.float32),
          pltpu.VMEM((CHUNK, D), jnp.float32),
          pltpu.VMEM((CHUNK, D), jnp.float32),
          pltpu.VMEM((NT, bpw), jnp.int32),
          pltpu.VMEM((NG, bpw), jnp.int32),
          pltpu.SemaphoreType.DMA,
      ],
  )
  def sc_kernel(g_idx_t, t_idx_t, genre_table, tag_lin,
                g_out, t_out,
                rows, g_acc, t_acc, idx_t, idx_g, sem):
    wid = lax.axis_index("s") * NC + lax.axis_index("c")
    base = wid * bpw

    pltpu.sync_copy(g_idx_t.at[:, pl.ds(base, bpw)], idx_g)
    pltpu.sync_copy(t_idx_t.at[:, pl.ds(base, bpw)], idx_t)

    def chunk_body(c, carry):
      off = c * CHUNK
      start = base + off

      hs = []
      for j in range(NG):
        hs.append(pltpu.async_copy(
            genre_table.at[idx_g.at[j, pl.ds(off, CHUNK)]],
            rows.at[pl.ds(j * CHUNK, CHUNK), :], sem))
      for h in hs:
        h.wait()
      _accumulate(rows, g_acc, NG, CHUNK)
      pltpu.sync_copy(g_acc, g_out.at[pl.ds(start, CHUNK)])

      hs = []
      for j in range(NT):
        hs.append(pltpu.async_copy(
            tag_lin.at[idx_t.at[j, pl.ds(off, CHUNK)]],
            rows.at[pl.ds(j * CHUNK, CHUNK), :], sem))
      for h in hs:
        h.wait()
      _accumulate(rows, t_acc, NT, CHUNK)
      pltpu.sync_copy(t_acc, t_out.at[pl.ds(start, CHUNK)])
      return carry

    lax.fori_loop(0, nch, chunk_body, 0)

  return sc_kernel


def _mlp_body(gidx_ref, tidx_ref, m_ref, g_ref, y_ref, t_ref,
              w1a_ref, w1b_ref, w1c_ref, w1d_ref, b1_ref,
              w2_ref, b2_ref, w3_ref, b3_ref, out_ref):
  cnt_g = jnp.sum((gidx_ref[...] != 0).astype(jnp.float32), axis=1,
                  keepdims=True)
  cnt_t = jnp.sum((tidx_ref[...] != 0).astype(jnp.float32), axis=1,
                  keepdims=True)
  g = g_ref[...] / jnp.maximum(cnt_g, 1e-9)
  t = t_ref[...] / jnp.maximum(cnt_t, 1e-9)
  f32 = jnp.float32
  x = (jnp.dot(m_ref[...], w1a_ref[...], preferred_element_type=f32)
       + jnp.dot(g, w1b_ref[...], preferred_element_type=f32)
       + jnp.dot(y_ref[...], w1c_ref[...], preferred_element_type=f32)
       + jnp.dot(t, w1d_ref[...], preferred_element_type=f32)
       + b1_ref[...])
  x = jnp.maximum(x, 0.0)
  x = jnp.maximum(jnp.dot(x, w2_ref[...], preferred_element_type=f32)
                  + b2_ref[...], 0.0)
  out_ref[...] = lax.dot_general(
      w3_ref[...], x, dimension_numbers=(((0,), (1,)), ((), ())),
      preferred_element_type=f32) + b3_ref[...]


def _mlp(B, H, gidx, tidx, m, g_sum, y, t_sum, W1, b1, W2, b2, W3, b3):
  R = 2048
  grid = B // R
  w1a = W1[:D]
  w1b = W1[D:2 * D]
  w1c = W1[2 * D:2 * D + YD]
  w1d = W1[2 * D + YD:]
  row = lambda i: (i, 0)
  rep = lambda i: (0, 0)
  return pl.pallas_call(
      _mlp_body,
      grid=(grid,),
      in_specs=[
          pl.BlockSpec((R, NG), row),
          pl.BlockSpec((R, NT), row),
          pl.BlockSpec((R, D), row),
          pl.BlockSpec((R, D), row),
          pl.BlockSpec((R, YD), row),
          pl.BlockSpec((R, D), row),
          pl.BlockSpec(w1a.shape, rep),
          pl.BlockSpec(w1b.shape, rep),
          pl.BlockSpec(w1c.shape, rep),
          pl.BlockSpec(w1d.shape, rep),
          pl.BlockSpec((1, H), rep),
          pl.BlockSpec(W2.shape, rep),
          pl.BlockSpec((1, H // 2), rep),
          pl.BlockSpec(W3.shape, rep),
          pl.BlockSpec((D, 1), rep),
      ],
      out_specs=pl.BlockSpec((D, R), lambda i: (0, i)),
      out_shape=jax.ShapeDtypeStruct((D, B), jnp.float32),
  )(gidx, tidx, m, g_sum, y, t_sum, w1a, w1b, w1c, w1d,
    b1.reshape(1, H), W2, b2.reshape(1, H // 2), W3,
    b3.reshape(D, 1)).T


def kernel(movie_id, padded_genre_indices, year_idx, padded_tag_indices,
           movie_table, genre_table, tag_table, year_table,
           W1, b1, W2, b2, W3, b3):
  B = movie_id.shape[0]
  H = W1.shape[1]
  mid = movie_id.astype(jnp.int32)
  gid_t = padded_genre_indices.astype(jnp.int32).T
  yid = year_idx.astype(jnp.int32)
  mid_lin = _remap(mid, M_SPLIT)
  tid_lin_t = _remap(padded_tag_indices.astype(jnp.int32), T_SPLIT).T

  tag_lin = _table_convert(tag_table.T, T_GRID)
  movie_lin = _table_convert(movie_table.T, M_GRID)

  g_sum, t_sum = _sc_gather_pool(B)(gid_t, tid_lin_t, genre_table, tag_lin)
  m_rows, y_rows = _sc_rows(B)(mid_lin, yid, movie_lin, year_table)

  return _mlp(B, H, padded_genre_indices.astype(jnp.int32),
              padded_tag_indices.astype(jnp.int32),
              m_rows, g_sum, y_rows, t_sum, W1, b1, W2, b2, W3, b3)
